# baseline (speedup 1.0000x reference)
import functools

import jax
import jax.numpy as jnp
from jax import lax
from jax.experimental import pallas as pl
from jax.experimental.pallas import tpu as pltpu
from jax.experimental.pallas import tpu_sc as plsc

VOCAB = 1000000
EMB_DIM = 64
FIX_LEN = 32
BATCH = 16384
H1 = 128
OUT = 2

NIDX = BATCH * FIX_LEN

_INFO = plsc.get_sparse_core_info()
_NC = _INFO.num_cores
_NS = _INFO.num_subcores
_NW = _NC * _NS
_BPW = BATCH // _NW
_IPW = _BPW * FIX_LEN
_CHUNK = 128
_N_CHUNKS = _IPW // _CHUNK
_NBUF = 4
_N_GROUPS = _N_CHUNKS // _NBUF


@functools.partial(
    pl.kernel,
    mesh=plsc.VectorSubcoreMesh(core_axis_name="c", subcore_axis_name="s"),
    out_type=jax.ShapeDtypeStruct((NIDX // 2, 128), jnp.float32),
    scratch_types=[
        pltpu.VMEM((FIX_LEN, _BPW), jnp.int32),
        pltpu.VMEM((_IPW,), jnp.int32),
        pltpu.VMEM((_IPW,), jnp.int32),
        pltpu.VMEM((_NBUF, _CHUNK, 128), jnp.float32),
        pltpu.SemaphoreType.DMA,
        pltpu.SemaphoreType.DMA,
        pltpu.SemaphoreType.DMA,
    ],
    compiler_params=pltpu.CompilerParams(
        use_tc_tiling_on_sc=True, needs_layout_passes=False
    ),
)
def _sc_gather(tflat_hbm, table2_hbm, out_hbm, tstage, idxv, hvv, rows, ssem,
               gsem, wsem):
    wid = lax.axis_index("s") * _NC + lax.axis_index("c")
    b0 = wid * _BPW
    base = wid * _IPW

    stage = [
        pltpu.async_copy(
            tflat_hbm.at[pl.ds(l * BATCH + b0, _BPW)], tstage.at[l], ssem
        )
        for l in range(FIX_LEN)
    ]
    for d in stage:
        d.wait()

    def reorder_group(k, carry):
        p0 = k * 16
        pv = jax.lax.iota(jnp.int32, 16) + p0
        li = jax.lax.rem(pv, FIX_LEN)
        bi = jax.lax.div(pv, FIX_LEN)
        v = plsc.load_gather(tstage, [li, bi])
        idxv[pl.ds(p0, 16)] = v >> 1
        hvv[pl.ds(p0, 16)] = v & 1
        return carry

    lax.fori_loop(0, _IPW // 16, reorder_group, 0)

    def start_gather(c, b):
        return pltpu.async_copy(
            table2_hbm.at[idxv.at[pl.ds(pl.multiple_of(c * _CHUNK, _CHUNK), _CHUNK)]],
            rows.at[b],
            gsem,
        )

    iota16 = jax.lax.iota(jnp.int32, 16)
    dri = iota16 >> 1
    dci0 = (iota16 & 1) * 64

    def compact_chunk(c, b):
        buf = rows.at[b]

        def block(bk, carry):
            rowv = iota16 + bk * 16
            hv = hvv[pl.ds(pl.multiple_of(c * _CHUNK + bk * 16, 16), 16)]
            sci0 = hv * 64
            drib = dri + bk * 8
            for w in range(EMB_DIM):
                g = plsc.load_gather(buf, [rowv, sci0 + w])
                plsc.store_scatter(buf, [drib, dci0 + w], g)
            return carry

        lax.fori_loop(0, _CHUNK // 16, block, 0)

    def group(g, carry):
        c0 = g * _NBUF
        gds = [start_gather(c0 + b, b) for b in range(_NBUF)]
        wds = []
        for b in range(_NBUF):
            gds[b].wait()
            compact_chunk(c0 + b, b)
            wds.append(
                pltpu.async_copy(
                    rows.at[b].at[pl.ds(0, _CHUNK // 2)],
                    out_hbm.at[
                        pl.ds(
                            pl.multiple_of(
                                (base + (c0 + b) * _CHUNK) // 2, _CHUNK // 2
                            ),
                            _CHUNK // 2,
                        )
                    ],
                    wsem,
                )
            )
        for wd in wds:
            wd.wait()
        return carry

    lax.fori_loop(0, _N_GROUPS, group, 0)


_TB = 512
_ROWS_PER_TB = _TB * FIX_LEN * EMB_DIM // 128


def _mlp_body(e_ref, w1_ref, b1_ref, w2_ref, b2_ref, o_ref):
    x = e_ref[...].reshape(_TB, FIX_LEN * EMB_DIM)
    h = jnp.dot(x, w1_ref[...], preferred_element_type=jnp.float32)
    h = h + b1_ref[...]
    h = jnp.where(h >= 0, h, 0.01 * h)
    o_ref[...] = (
        jnp.dot(h, w2_ref[...], preferred_element_type=jnp.float32) + b2_ref[...]
    )


def kernel(texts, emb_table, W1, b1, W2, b2):
    tflat = texts.T.reshape(-1).astype(jnp.int32)
    t2 = emb_table.reshape(VOCAB // 2, 2 * EMB_DIM)
    e2 = _sc_gather(tflat, t2)

    out = pl.pallas_call(
        _mlp_body,
        grid=(BATCH // _TB,),
        in_specs=[
            pl.BlockSpec((_ROWS_PER_TB, 128), lambda i: (i, 0)),
            pl.BlockSpec((FIX_LEN * EMB_DIM, H1), lambda i: (0, 0)),
            pl.BlockSpec((1, H1), lambda i: (0, 0)),
            pl.BlockSpec((H1, OUT), lambda i: (0, 0)),
            pl.BlockSpec((1, OUT), lambda i: (0, 0)),
        ],
        out_specs=pl.BlockSpec((_TB, OUT), lambda i: (i, 0)),
        out_shape=jax.ShapeDtypeStruct((BATCH, OUT), jnp.float32),
    )(e2, W1, b1.reshape(1, H1), W2, b2.reshape(1, OUT))
    return out


# speedup vs baseline: 1.9442x; 1.9442x over previous
import functools

import jax
import jax.numpy as jnp
from jax import lax
from jax.experimental import pallas as pl
from jax.experimental.pallas import tpu as pltpu
from jax.experimental.pallas import tpu_sc as plsc

VOCAB = 1000000
EMB_DIM = 64
FIX_LEN = 32
BATCH = 16384
H1 = 128
OUT = 2

NIDX = BATCH * FIX_LEN

_INFO = plsc.get_sparse_core_info()
_NC = _INFO.num_cores
_NS = _INFO.num_subcores
_NW = _NC * _NS
_BPW = BATCH // _NW
_IPW = _BPW * FIX_LEN
_CHUNK = 128
_N_CHUNKS = _IPW // _CHUNK
_NBUF = 4
_N_GROUPS = _N_CHUNKS // _NBUF


@functools.partial(
    pl.kernel,
    mesh=plsc.VectorSubcoreMesh(core_axis_name="c", subcore_axis_name="s"),
    out_type=jax.ShapeDtypeStruct((NIDX // 2, 128), jnp.float32),
    scratch_types=[
        pltpu.VMEM((FIX_LEN, _BPW), jnp.int32),
        pltpu.VMEM((_IPW,), jnp.int32),
        pltpu.VMEM((_IPW,), jnp.int32),
        pltpu.VMEM((_NBUF, _CHUNK, 128), jnp.float32),
        pltpu.SemaphoreType.DMA,
        pltpu.SemaphoreType.DMA,
        pltpu.SemaphoreType.DMA,
    ],
    compiler_params=pltpu.CompilerParams(
        use_tc_tiling_on_sc=True, needs_layout_passes=False
    ),
)
def _sc_gather(tflat_hbm, table2_hbm, out_hbm, tstage, idxv, hvv, rows, ssem,
               gsem, wsem):
    wid = lax.axis_index("s") * _NC + lax.axis_index("c")
    b0 = wid * _BPW
    base = wid * _IPW

    stage = [
        pltpu.async_copy(
            tflat_hbm.at[pl.ds(l * BATCH + b0, _BPW)], tstage.at[l], ssem
        )
        for l in range(FIX_LEN)
    ]
    for d in stage:
        d.wait()

    def reorder_group(k, carry):
        p0 = k * 16
        pv = jax.lax.iota(jnp.int32, 16) + p0
        li = jax.lax.rem(pv, FIX_LEN)
        bi = jax.lax.div(pv, FIX_LEN)
        v = plsc.load_gather(tstage, [li, bi])
        idxv[pl.ds(p0, 16)] = v >> 1
        hvv[pl.ds(p0, 16)] = v & 1
        return carry

    lax.fori_loop(0, _IPW // 16, reorder_group, 0)

    def start_gather(c, b):
        return pltpu.async_copy(
            table2_hbm.at[idxv.at[pl.ds(pl.multiple_of(c * _CHUNK, _CHUNK), _CHUNK)]],
            rows.at[b],
            gsem,
        )

    def compact_chunk(c, b):
        def block(bk, carry):
            j0 = bk * 16
            hv = hvv[pl.ds(pl.multiple_of(c * _CHUNK + j0, 16), 16)]
            for i in range(16):
                j = j0 + i
                soff = hv[i] * 64
                doff = (i & 1) * 64
                src = rows.at[b, j]
                dst = rows.at[b, j >> 1]
                for g_ in range(4):
                    dst[pl.ds(doff + g_ * 16, 16)] = src[pl.ds(soff + g_ * 16, 16)]
            return carry

        lax.fori_loop(0, _CHUNK // 16, block, 0)

    def group(g, carry):
        c0 = g * _NBUF
        gds = [start_gather(c0 + b, b) for b in range(_NBUF)]
        wds = []
        for b in range(_NBUF):
            gds[b].wait()
            compact_chunk(c0 + b, b)
            wds.append(
                pltpu.async_copy(
                    rows.at[b].at[pl.ds(0, _CHUNK // 2)],
                    out_hbm.at[
                        pl.ds(
                            pl.multiple_of(
                                (base + (c0 + b) * _CHUNK) // 2, _CHUNK // 2
                            ),
                            _CHUNK // 2,
                        )
                    ],
                    wsem,
                )
            )
        for wd in wds:
            wd.wait()
        return carry

    lax.fori_loop(0, _N_GROUPS, group, 0)


_TB = 512
_ROWS_PER_TB = _TB * FIX_LEN * EMB_DIM // 128


def _mlp_body(e_ref, w1_ref, b1_ref, w2_ref, b2_ref, o_ref):
    x = e_ref[...].reshape(_TB, FIX_LEN * EMB_DIM)
    h = jnp.dot(x, w1_ref[...], preferred_element_type=jnp.float32)
    h = h + b1_ref[...]
    h = jnp.where(h >= 0, h, 0.01 * h)
    o_ref[...] = (
        jnp.dot(h, w2_ref[...], preferred_element_type=jnp.float32) + b2_ref[...]
    )


def kernel(texts, emb_table, W1, b1, W2, b2):
    tflat = texts.T.reshape(-1).astype(jnp.int32)
    t2 = emb_table.reshape(VOCAB // 2, 2 * EMB_DIM)
    e2 = _sc_gather(tflat, t2)

    out = pl.pallas_call(
        _mlp_body,
        grid=(BATCH // _TB,),
        in_specs=[
            pl.BlockSpec((_ROWS_PER_TB, 128), lambda i: (i, 0)),
            pl.BlockSpec((FIX_LEN * EMB_DIM, H1), lambda i: (0, 0)),
            pl.BlockSpec((1, H1), lambda i: (0, 0)),
            pl.BlockSpec((H1, OUT), lambda i: (0, 0)),
            pl.BlockSpec((1, OUT), lambda i: (0, 0)),
        ],
        out_specs=pl.BlockSpec((_TB, OUT), lambda i: (i, 0)),
        out_shape=jax.ShapeDtypeStruct((BATCH, OUT), jnp.float32),
    )(e2, W1, b1.reshape(1, H1), W2, b2.reshape(1, OUT))
    return out


# speedup vs baseline: 2.5313x; 1.3020x over previous
import functools

import jax
import jax.numpy as jnp
from jax import lax
from jax.experimental import pallas as pl
from jax.experimental.pallas import tpu as pltpu
from jax.experimental.pallas import tpu_sc as plsc

VOCAB = 1000000
EMB_DIM = 64
FIX_LEN = 32
BATCH = 16384
H1 = 128
OUT = 2

NIDX = BATCH * FIX_LEN

_INFO = plsc.get_sparse_core_info()
_NC = _INFO.num_cores
_NS = _INFO.num_subcores
_NW = _NC * _NS
_BPW = BATCH // _NW
_IPW = _BPW * FIX_LEN
_CHUNK = 256
_N_CHUNKS = _IPW // _CHUNK
_NBUF = 4


@functools.partial(
    pl.kernel,
    mesh=plsc.VectorSubcoreMesh(core_axis_name="c", subcore_axis_name="s"),
    out_type=jax.ShapeDtypeStruct((NIDX, EMB_DIM), jnp.float32),
    scratch_types=[
        pltpu.VMEM((FIX_LEN, _BPW), jnp.int32),
        pltpu.VMEM((_IPW,), jnp.int32),
        pltpu.VMEM((_NBUF, _CHUNK, EMB_DIM), jnp.float32),
        pltpu.SemaphoreType.DMA,
        pltpu.SemaphoreType.DMA,
        pltpu.SemaphoreType.DMA,
    ],
    compiler_params=pltpu.CompilerParams(
        use_tc_tiling_on_sc=False, needs_layout_passes=False
    ),
)
def _sc_gather(tflat_hbm, table_hbm, out_hbm, tstage, idxv, rows, ssem, gsem,
               wsem):
    wid = lax.axis_index("s") * _NC + lax.axis_index("c")
    b0 = wid * _BPW
    base = wid * _IPW

    stage = [
        pltpu.async_copy(
            tflat_hbm.at[pl.ds(l * BATCH + b0, _BPW)], tstage.at[l], ssem
        )
        for l in range(FIX_LEN)
    ]
    for d in stage:
        d.wait()

    def reorder_group(k, carry):
        p0 = k * 16
        pv = jax.lax.iota(jnp.int32, 16) + p0
        li = jax.lax.rem(pv, FIX_LEN)
        bi = jax.lax.div(pv, FIX_LEN)
        idxv[pl.ds(p0, 16)] = plsc.load_gather(tstage, [li, bi])
        return carry

    def reorder_chunk(c):
        lax.fori_loop(c * _CHUNK // 16, (c + 1) * _CHUNK // 16, reorder_group, 0)

    def start_gather(c):
        return pltpu.async_copy(
            table_hbm.at[idxv.at[pl.ds(c * _CHUNK, _CHUNK)]],
            rows.at[c % _NBUF],
            gsem,
        )

    gds = [None] * _N_CHUNKS
    wds = [None] * _N_CHUNKS
    for c in range(_NBUF - 1):
        reorder_chunk(c)
        gds[c] = start_gather(c)
    for c in range(_N_CHUNKS):
        n = c + _NBUF - 1
        if n < _N_CHUNKS:
            reorder_chunk(n)
        gds[c].wait()
        if n < _N_CHUNKS:
            if c - 1 >= 0:
                wds[c - 1].wait()
            gds[n] = start_gather(n)
        wds[c] = pltpu.async_copy(
            rows.at[c % _NBUF], out_hbm.at[pl.ds(base + c * _CHUNK, _CHUNK)],
            wsem,
        )
    for c in range(_N_CHUNKS - _NBUF, _N_CHUNKS):
        wds[c].wait()


_TB = 512
_ROWS_PER_TB = _TB * FIX_LEN * EMB_DIM // 128


def _mlp_body(e_ref, w1_ref, b1_ref, w2_ref, b2_ref, o_ref):
    x = e_ref[...].reshape(_TB, FIX_LEN * EMB_DIM)
    h = jnp.dot(x, w1_ref[...], preferred_element_type=jnp.float32)
    h = h + b1_ref[...]
    h = jnp.where(h >= 0, h, 0.01 * h)
    o_ref[...] = (
        jnp.dot(h, w2_ref[...], preferred_element_type=jnp.float32) + b2_ref[...]
    )


def kernel(texts, emb_table, W1, b1, W2, b2):
    tflat = texts.T.reshape(-1).astype(jnp.int32)
    embeds = _sc_gather(tflat, emb_table)
    e2 = embeds.reshape(NIDX // 2, 128)

    out = pl.pallas_call(
        _mlp_body,
        grid=(BATCH // _TB,),
        in_specs=[
            pl.BlockSpec((_ROWS_PER_TB, 128), lambda i: (i, 0)),
            pl.BlockSpec((FIX_LEN * EMB_DIM, H1), lambda i: (0, 0)),
            pl.BlockSpec((1, H1), lambda i: (0, 0)),
            pl.BlockSpec((H1, OUT), lambda i: (0, 0)),
            pl.BlockSpec((1, OUT), lambda i: (0, 0)),
        ],
        out_specs=pl.BlockSpec((_TB, OUT), lambda i: (i, 0)),
        out_shape=jax.ShapeDtypeStruct((BATCH, OUT), jnp.float32),
    )(e2, W1, b1.reshape(1, H1), W2, b2.reshape(1, OUT))
    return out


# speedup vs baseline: 2.5617x; 1.0120x over previous
import functools

import jax
import jax.numpy as jnp
from jax import lax
from jax.experimental import pallas as pl
from jax.experimental.pallas import tpu as pltpu
from jax.experimental.pallas import tpu_sc as plsc

VOCAB = 1000000
EMB_DIM = 64
FIX_LEN = 32
BATCH = 16384
H1 = 128
OUT = 2

NIDX = BATCH * FIX_LEN

_INFO = plsc.get_sparse_core_info()
_NC = _INFO.num_cores
_NS = _INFO.num_subcores
_NW = _NC * _NS
_BPW = BATCH // _NW
_IPW = _BPW * FIX_LEN
_CHUNK = 256
_N_CHUNKS = _IPW // _CHUNK
_NBUF = 4


@functools.partial(
    pl.kernel,
    mesh=plsc.VectorSubcoreMesh(core_axis_name="c", subcore_axis_name="s"),
    out_type=jax.ShapeDtypeStruct((NIDX, EMB_DIM), jnp.float32),
    scratch_types=[
        pltpu.VMEM((FIX_LEN, _BPW), jnp.int32),
        pltpu.VMEM((_IPW,), jnp.int32),
        pltpu.VMEM((_NBUF, _CHUNK, EMB_DIM), jnp.float32),
        pltpu.SemaphoreType.DMA,
        pltpu.SemaphoreType.DMA,
        pltpu.SemaphoreType.DMA,
    ],
    compiler_params=pltpu.CompilerParams(
        use_tc_tiling_on_sc=False, needs_layout_passes=False
    ),
)
def _sc_gather(tflat_hbm, table_hbm, out_hbm, tstage, idxv, rows, ssem, gsem,
               wsem):
    wid = lax.axis_index("s") * _NC + lax.axis_index("c")
    b0 = wid * _BPW
    base = wid * _IPW

    stage = [
        pltpu.async_copy(
            tflat_hbm.at[pl.ds(l * BATCH + b0, _BPW)], tstage.at[l], ssem
        )
        for l in range(FIX_LEN)
    ]
    for d in stage:
        d.wait()

    def reorder_group(k, carry):
        p0 = k * 16
        pv = jax.lax.iota(jnp.int32, 16) + p0
        li = jax.lax.rem(pv, FIX_LEN)
        bi = jax.lax.div(pv, FIX_LEN)
        idxv[pl.ds(p0, 16)] = plsc.load_gather(tstage, [li, bi])
        return carry

    def reorder_chunk(c):
        lax.fori_loop(c * _CHUNK // 16, (c + 1) * _CHUNK // 16, reorder_group, 0)

    def start_gather(c):
        return pltpu.async_copy(
            table_hbm.at[idxv.at[pl.ds(c * _CHUNK, _CHUNK)]],
            rows.at[c % _NBUF],
            gsem,
        )

    gds = [None] * _N_CHUNKS
    wds = [None] * _N_CHUNKS
    for c in range(_NBUF - 1):
        reorder_chunk(c)
        gds[c] = start_gather(c)
    for c in range(_N_CHUNKS):
        n = c + _NBUF - 1
        if n < _N_CHUNKS:
            reorder_chunk(n)
        gds[c].wait()
        if n < _N_CHUNKS:
            if c - 1 >= 0:
                wds[c - 1].wait()
            gds[n] = start_gather(n)
        wds[c] = pltpu.async_copy(
            rows.at[c % _NBUF], out_hbm.at[pl.ds(base + c * _CHUNK, _CHUNK)],
            wsem,
        )
    for c in range(_N_CHUNKS - _NBUF, _N_CHUNKS):
        wds[c].wait()


_TB = 1024
_ROWS_PER_TB = _TB * FIX_LEN * EMB_DIM // 128


def _mlp_body(e_ref, w1_ref, b1_ref, w2_ref, b2_ref, o_ref):
    x = e_ref[...].reshape(_TB, FIX_LEN * EMB_DIM)
    h = jnp.dot(x, w1_ref[...], preferred_element_type=jnp.float32)
    h = h + b1_ref[...]
    h = jnp.where(h >= 0, h, 0.01 * h)
    o_ref[...] = (
        jnp.dot(h, w2_ref[...], preferred_element_type=jnp.float32) + b2_ref[...]
    )


def kernel(texts, emb_table, W1, b1, W2, b2):
    tflat = texts.T.reshape(-1).astype(jnp.int32)
    embeds = _sc_gather(tflat, emb_table)
    e2 = embeds.reshape(NIDX // 2, 128)

    out = pl.pallas_call(
        _mlp_body,
        grid=(BATCH // _TB,),
        in_specs=[
            pl.BlockSpec((_ROWS_PER_TB, 128), lambda i: (i, 0)),
            pl.BlockSpec((FIX_LEN * EMB_DIM, H1), lambda i: (0, 0)),
            pl.BlockSpec((1, H1), lambda i: (0, 0)),
            pl.BlockSpec((H1, OUT), lambda i: (0, 0)),
            pl.BlockSpec((1, OUT), lambda i: (0, 0)),
        ],
        out_specs=pl.BlockSpec((_TB, OUT), lambda i: (i, 0)),
        out_shape=jax.ShapeDtypeStruct((BATCH, OUT), jnp.float32),
    )(e2, W1, b1.reshape(1, H1), W2, b2.reshape(1, OUT))
    return out


# speedup vs baseline: 2.5669x; 1.0020x over previous
import functools

import jax
import jax.numpy as jnp
from jax import lax
from jax.experimental import pallas as pl
from jax.experimental.pallas import tpu as pltpu
from jax.experimental.pallas import tpu_sc as plsc

VOCAB = 1000000
EMB_DIM = 64
FIX_LEN = 32
BATCH = 16384
H1 = 128
OUT = 2

NIDX = BATCH * FIX_LEN

_INFO = plsc.get_sparse_core_info()
_NC = _INFO.num_cores
_NS = _INFO.num_subcores
_NW = _NC * _NS
_BPW = BATCH // _NW
_IPW = _BPW * FIX_LEN
_CHUNK = 256
_N_CHUNKS = _IPW // _CHUNK
_NBUF = 4


@functools.partial(
    pl.kernel,
    mesh=plsc.VectorSubcoreMesh(core_axis_name="c", subcore_axis_name="s"),
    out_type=jax.ShapeDtypeStruct((NIDX, EMB_DIM), jnp.float32),
    scratch_types=[
        pltpu.VMEM((FIX_LEN, _BPW), jnp.int32),
        pltpu.VMEM((_IPW,), jnp.int32),
        pltpu.VMEM((_NBUF, _CHUNK, EMB_DIM), jnp.float32),
        pltpu.SemaphoreType.DMA,
        pltpu.SemaphoreType.DMA,
        pltpu.SemaphoreType.DMA,
    ],
    compiler_params=pltpu.CompilerParams(
        use_tc_tiling_on_sc=False, needs_layout_passes=False
    ),
)
def _sc_gather(tflat_hbm, table_hbm, out_hbm, tstage, idxv, rows, ssem, gsem,
               wsem):
    wid = lax.axis_index("s") * _NC + lax.axis_index("c")
    b0 = wid * _BPW
    base = wid * _IPW

    stage = [
        pltpu.async_copy(
            tflat_hbm.at[pl.ds(l * BATCH + b0, _BPW)], tstage.at[l], ssem
        )
        for l in range(FIX_LEN)
    ]
    for d in stage:
        d.wait()

    def reorder_group(k, carry):
        p0 = k * 16
        pv = jax.lax.iota(jnp.int32, 16) + p0
        li = jax.lax.rem(pv, FIX_LEN)
        bi = jax.lax.div(pv, FIX_LEN)
        idxv[pl.ds(p0, 16)] = plsc.load_gather(tstage, [li, bi])
        return carry

    def reorder_chunk(c):
        lax.fori_loop(c * _CHUNK // 16, (c + 1) * _CHUNK // 16, reorder_group, 0)

    def start_gather(c):
        return pltpu.async_copy(
            table_hbm.at[idxv.at[pl.ds(c * _CHUNK, _CHUNK)]],
            rows.at[c % _NBUF],
            gsem,
        )

    gds = [None] * _N_CHUNKS
    wds = [None] * _N_CHUNKS
    for c in range(_NBUF - 1):
        reorder_chunk(c)
        gds[c] = start_gather(c)
    for c in range(_N_CHUNKS):
        n = c + _NBUF - 1
        if n < _N_CHUNKS:
            reorder_chunk(n)
        gds[c].wait()
        if n < _N_CHUNKS:
            if c - 1 >= 0:
                wds[c - 1].wait()
            gds[n] = start_gather(n)
        wds[c] = pltpu.async_copy(
            rows.at[c % _NBUF], out_hbm.at[pl.ds(base + c * _CHUNK, _CHUNK)],
            wsem,
        )
    for c in range(_N_CHUNKS - _NBUF, _N_CHUNKS):
        wds[c].wait()


_TB = 2048
_ROWS_PER_TB = _TB * FIX_LEN * EMB_DIM // 128


def _mlp_body(e_ref, w1_ref, b1_ref, w2_ref, b2_ref, o_ref):
    x = e_ref[...].reshape(_TB, FIX_LEN * EMB_DIM)
    h = jnp.dot(x, w1_ref[...], preferred_element_type=jnp.float32)
    h = h + b1_ref[...]
    h = jnp.where(h >= 0, h, 0.01 * h)
    o_ref[...] = (
        jnp.dot(h, w2_ref[...], preferred_element_type=jnp.float32) + b2_ref[...]
    )


def kernel(texts, emb_table, W1, b1, W2, b2):
    tflat = texts.T.reshape(-1).astype(jnp.int32)
    embeds = _sc_gather(tflat, emb_table)
    e2 = embeds.reshape(NIDX // 2, 128)

    out = pl.pallas_call(
        _mlp_body,
        grid=(BATCH // _TB,),
        in_specs=[
            pl.BlockSpec((_ROWS_PER_TB, 128), lambda i: (i, 0)),
            pl.BlockSpec((FIX_LEN * EMB_DIM, H1), lambda i: (0, 0)),
            pl.BlockSpec((1, H1), lambda i: (0, 0)),
            pl.BlockSpec((H1, OUT), lambda i: (0, 0)),
            pl.BlockSpec((1, OUT), lambda i: (0, 0)),
        ],
        out_specs=pl.BlockSpec((_TB, OUT), lambda i: (i, 0)),
        out_shape=jax.ShapeDtypeStruct((BATCH, OUT), jnp.float32),
    )(e2, W1, b1.reshape(1, H1), W2, b2.reshape(1, OUT))
    return out
